# baseline (speedup 1.0000x reference)
import jax
import jax.numpy as jnp
from jax.experimental import pallas as pl
from jax.experimental.pallas import tpu as pltpu

_ROWS = 1024
_COLS = 27278
_BLOCK_ROWS = 16
_STEPS = _ROWS // _BLOCK_ROWS
_NBUF = 8


def _copy(hbm_ref, buf_ref, sem, chunk, slot):
    return pltpu.make_async_copy(
        hbm_ref.at[pl.ds(chunk * _BLOCK_ROWS, _BLOCK_ROWS), :],
        buf_ref.at[slot],
        sem.at[slot],
    )


def _masked_mse_body(o_hbm, t_hbm, res_ref,
                     o_bufs, t_bufs, acc_ref, cnt_ref, o_sems, t_sems):
    i = pl.program_id(0)
    slot = jax.lax.rem(i, _NBUF)

    @pl.when(i == 0)
    def _warmup():
        acc_ref[...] = jnp.zeros_like(acc_ref)
        cnt_ref[...] = jnp.zeros_like(cnt_ref)
        for s in range(_NBUF):
            _copy(o_hbm, o_bufs, o_sems, s, s).start()
            _copy(t_hbm, t_bufs, t_sems, s, s).start()

    _copy(o_hbm, o_bufs, o_sems, i, slot).wait()
    _copy(t_hbm, t_bufs, t_sems, i, slot).wait()

    o = o_bufs[slot]
    t = t_bufs[slot]
    m = t != -1.0
    d = o - t
    acc_ref[...] += jnp.where(m, d * d, 0.0)
    cnt_ref[...] += m.astype(jnp.float32)

    nxt = i + _NBUF

    @pl.when(nxt < _STEPS)
    def _prefetch():
        _copy(o_hbm, o_bufs, o_sems, nxt, slot).start()
        _copy(t_hbm, t_bufs, t_sems, nxt, slot).start()

    @pl.when(i == _STEPS - 1)
    def _fin():
        res_ref[0, 0] = jnp.sum(acc_ref[...]) / jnp.sum(cnt_ref[...])


def kernel(output, target):
    res = pl.pallas_call(
        _masked_mse_body,
        grid=(_STEPS,),
        in_specs=[
            pl.BlockSpec(memory_space=pl.ANY),
            pl.BlockSpec(memory_space=pl.ANY),
        ],
        out_specs=pl.BlockSpec(memory_space=pltpu.SMEM),
        out_shape=jax.ShapeDtypeStruct((1, 1), jnp.float32),
        scratch_shapes=[
            pltpu.VMEM((_NBUF, _BLOCK_ROWS, _COLS), jnp.float32),
            pltpu.VMEM((_NBUF, _BLOCK_ROWS, _COLS), jnp.float32),
            pltpu.VMEM((_BLOCK_ROWS, _COLS), jnp.float32),
            pltpu.VMEM((_BLOCK_ROWS, _COLS), jnp.float32),
            pltpu.SemaphoreType.DMA((_NBUF,)),
            pltpu.SemaphoreType.DMA((_NBUF,)),
        ],
    )(output, target)
    return res.reshape(())


# speedup vs baseline: 3.4885x; 3.4885x over previous
import jax
import jax.numpy as jnp
from jax.experimental import pallas as pl
from jax.experimental.pallas import tpu as pltpu

_ROWS = 27278
_COLS = 1024
_BLOCK_ROWS = 1024
_STEPS = (_ROWS + _BLOCK_ROWS - 1) // _BLOCK_ROWS


def _masked_mse_body(o_ref, t_ref, res_ref, acc_ref, cnt_ref):
    i = pl.program_id(0)

    @pl.when(i == 0)
    def _init():
        acc_ref[...] = jnp.zeros_like(acc_ref)
        cnt_ref[...] = jnp.zeros_like(cnt_ref)

    o = o_ref[...]
    t = t_ref[...]
    m = t != -1.0
    d = o - t

    @pl.when(i < _STEPS - 1)
    def _full():
        acc_ref[...] += jnp.where(m, d * d, 0.0)
        cnt_ref[...] += m.astype(jnp.float32)

    @pl.when(i == _STEPS - 1)
    def _tail():
        rows_left = _ROWS - (_STEPS - 1) * _BLOCK_ROWS
        valid = jax.lax.broadcasted_iota(
            jnp.int32, (_BLOCK_ROWS, _COLS), 0) < rows_left
        mv = jnp.logical_and(m, valid)
        acc_ref[...] += jnp.where(mv, d * d, 0.0)
        cnt_ref[...] += mv.astype(jnp.float32)
        res_ref[0, 0] = jnp.sum(acc_ref[...]) / jnp.sum(cnt_ref[...])


def kernel(output, target):
    spec = pl.BlockSpec((_BLOCK_ROWS, _COLS), lambda i: (i, 0))
    res = pl.pallas_call(
        _masked_mse_body,
        grid=(_STEPS,),
        in_specs=[spec, spec],
        out_specs=pl.BlockSpec(memory_space=pltpu.SMEM),
        out_shape=jax.ShapeDtypeStruct((1, 1), jnp.float32),
        scratch_shapes=[
            pltpu.VMEM((_BLOCK_ROWS, _COLS), jnp.float32),
            pltpu.VMEM((_BLOCK_ROWS, _COLS), jnp.float32),
        ],
    )(output.T, target.T)
    return res.reshape(())
